# Initial kernel scaffold; baseline (speedup 1.0000x reference)
import jax
import jax.numpy as jnp
from jax.experimental import pallas as pl


def kernel(x, lin_in_W, lin_in_b, conv0_W1, conv0_b1, conv0_W2, conv0_b2, conv0_eps, conv1_W1, conv1_b1, conv1_W2, conv1_b2, conv1_eps, lin_out_W, lin_out_b, edge_index):
    raise NotImplementedError("write your pallas kernel here")


# speedup vs baseline: 238.3755x; 238.3755x over previous
import functools

import jax
import jax.numpy as jnp
import numpy as np
from jax import lax
from jax.experimental import pallas as pl
from jax.experimental.pallas import tpu as pltpu
from jax.experimental.pallas import tpu_sc as plsc

N = 10000
NPAD = 10240
E = 160000
EB = 640
F = 128
RB = 640
GCHUNK = 128
NTILES = 32

_HIGH = lax.Precision.HIGHEST


def _tmat():
    T = np.zeros((F, F), np.float32)
    for m in range(2):
        for i in range(8):
            for j in range(8):
                q = 64 * m + i * 8 + j
                p = 64 * m + j * 8 + i
                T[q, q] += 1.0
                T[p, q] -= 1.0
    return jnp.asarray(T)


def _bexpm(A, order, squarings):
    A = A * (1.0 / (1 << squarings))
    eye = (lax.broadcasted_iota(jnp.int32, (8, 8), 0)
           == lax.broadcasted_iota(jnp.int32, (8, 8), 1)
           ).astype(jnp.float32)[:, :, None]
    P = A * (1.0 / order) + eye
    for k in range(order - 1, 0, -1):
        M = jnp.sum(A[:, :, None, :] * P[None, :, :, :], axis=1)
        P = M * (1.0 / k) + eye
    for _ in range(squarings):
        P = jnp.sum(P[:, :, None, :] * P[None, :, :, :], axis=1)
    return P


def _bf(a):
    return a.astype(jnp.bfloat16).astype(jnp.float32)


def _bdot(a, b):
    return jnp.dot(a.astype(jnp.bfloat16), b.astype(jnp.bfloat16),
                   preferred_element_type=jnp.float32)


def _dense_body(hr_ref, hc_ref, w1a_ref, w1b_ref, b1_ref, w2_ref, b2_ref,
                t_ref, msg_ref, *, order, squarings):
    hr = hr_ref[...]
    hc = hc_ref[...]
    pre = (_bdot(hr, w1a_ref[...]) + _bdot(hc, w1b_ref[...]) + b1_ref[...])
    h1 = jnp.maximum(pre, 0.0)
    amaps = _bdot(h1, w2_ref[...]) + b2_ref[...]
    aall = jnp.dot(amaps, t_ref[...], precision=_HIGH)
    a_t = aall.T
    U = a_t[0:64].reshape(8, 8, EB)
    V = a_t[64:128].reshape(8, 8, EB)
    Fu = _bexpm(U, order, squarings)
    Fv = _bexpm(V, order, squarings)
    Fub = _bf(Fu)
    Fvb = _bf(Fv)
    hrT = _bf(hr.T.reshape(16, 8, EB))
    hcT = _bf(hc.T.reshape(16, 8, EB))
    p_i = jnp.sum(Fub[None, :, :, :] * hcT[:, None, :, :], axis=2)
    p_j = jnp.sum(Fvb[None, :, :, :] * hrT[:, None, :, :], axis=2)
    errb = _bf(p_i - p_j)
    msg3 = jnp.sum(Fub[None, :, :, :] * errb[:, :, None, :], axis=1)
    msg_ref[...] = msg3.reshape(F, EB).T


def _edge_dense(G, w1a, w1b, b1, w2, b2, tmat, order, squarings):
    nb = E // EB
    return pl.pallas_call(
        functools.partial(_dense_body, order=order, squarings=squarings),
        grid=(nb,),
        in_specs=[
            pl.BlockSpec((EB, F), lambda i: (i, 0)),
            pl.BlockSpec((EB, F), lambda i: (i + nb, 0)),
            pl.BlockSpec((F, 64), lambda i: (0, 0)),
            pl.BlockSpec((F, 64), lambda i: (0, 0)),
            pl.BlockSpec((1, 64), lambda i: (0, 0)),
            pl.BlockSpec((64, F), lambda i: (0, 0)),
            pl.BlockSpec((1, F), lambda i: (0, 0)),
            pl.BlockSpec((F, F), lambda i: (0, 0)),
        ],
        out_specs=pl.BlockSpec((EB, F), lambda i: (i, 0)),
        out_shape=jax.ShapeDtypeStruct((E, F), jnp.float32),
    )(G, G, w1a, w1b, b1, w2, b2, tmat)


def _linear_body(x_ref, w_ref, b_ref, o_ref):
    o_ref[...] = _bdot(x_ref[...], w_ref[...]) + b_ref[...]


def _lin(x, w, b):
    return pl.pallas_call(
        _linear_body,
        grid=(NPAD // RB,),
        in_specs=[
            pl.BlockSpec((RB, F), lambda i: (i, 0)),
            pl.BlockSpec((F, F), lambda i: (0, 0)),
            pl.BlockSpec((1, F), lambda i: (0, 0)),
        ],
        out_specs=pl.BlockSpec((RB, F), lambda i: (i, 0)),
        out_shape=jax.ShapeDtypeStruct((NPAD, F), jnp.float32),
    )(x, w, b)


def _elu(t):
    return jnp.where(t > 0.0, t, jnp.exp(jnp.minimum(t, 0.0)) - 1.0)


def _update_body(h_ref, p0_ref, p1_ref, eps_ref, o_ref):
    epsv = eps_ref[0:1, 0:1]
    t = h_ref[...] - epsv * (p0_ref[...] + p1_ref[...])
    o_ref[...] = _elu(t)


def _update(h, parts, eps_arr):
    nb = NPAD // RB
    return pl.pallas_call(
        _update_body,
        grid=(nb,),
        in_specs=[
            pl.BlockSpec((RB, F), lambda i: (i, 0)),
            pl.BlockSpec((RB, F), lambda i: (i, 0)),
            pl.BlockSpec((RB, F), lambda i: (i + nb, 0)),
            pl.BlockSpec((8, F), lambda i: (0, 0)),
        ],
        out_specs=pl.BlockSpec((RB, F), lambda i: (i, 0)),
        out_shape=jax.ShapeDtypeStruct((NPAD, F), jnp.float32),
    )(h, parts, parts, eps_arr)


def _update_linout_body(h_ref, p0_ref, p1_ref, eps_ref, w_ref, b_ref, o_ref):
    epsv = eps_ref[0:1, 0:1]
    t = _elu(h_ref[...] - epsv * (p0_ref[...] + p1_ref[...]))
    o_ref[...] = _bdot(t, w_ref[...]) + b_ref[...]


def _update_linout(h, parts, eps_arr, w, b):
    nb = NPAD // RB
    return pl.pallas_call(
        _update_linout_body,
        grid=(nb,),
        in_specs=[
            pl.BlockSpec((RB, F), lambda i: (i, 0)),
            pl.BlockSpec((RB, F), lambda i: (i, 0)),
            pl.BlockSpec((RB, F), lambda i: (i + nb, 0)),
            pl.BlockSpec((8, F), lambda i: (0, 0)),
            pl.BlockSpec((F, F), lambda i: (0, 0)),
            pl.BlockSpec((1, F), lambda i: (0, 0)),
        ],
        out_specs=pl.BlockSpec((RB, F), lambda i: (i, 0)),
        out_shape=jax.ShapeDtypeStruct((NPAD, F), jnp.float32),
    )(h, parts, parts, eps_arr, w, b)


def _sc_mesh():
    return plsc.VectorSubcoreMesh(core_axis_name="c", subcore_axis_name="s")


def _sc_gather(h, idx):
    nchunk = (2 * E) // GCHUNK

    @functools.partial(
        pl.kernel,
        out_type=jax.ShapeDtypeStruct((2 * E, F), jnp.float32),
        mesh=_sc_mesh(),
        scratch_types=[
            pltpu.VMEM((GCHUNK,), jnp.int32),
            pltpu.VMEM((GCHUNK, F), jnp.float32),
            pltpu.SemaphoreType.DMA,
        ],
    )
    def k(h_hbm, idx_hbm, out_hbm, idx_v, rows_v, sem):
        wid = lax.axis_index("s") * 2 + lax.axis_index("c")
        nloop = (nchunk + NTILES - 1) // NTILES

        @pl.loop(0, nloop)
        def _(c):
            cid = wid + NTILES * c

            @pl.when(cid < nchunk)
            def _():
                base = cid * GCHUNK
                pltpu.sync_copy(idx_hbm.at[pl.ds(base, GCHUNK)], idx_v)
                pltpu.async_copy(h_hbm.at[idx_v], rows_v, sem).wait()
                pltpu.sync_copy(rows_v, out_hbm.at[pl.ds(base, GCHUNK)])

    return k(h, idx)


def _sc_scatter(msg, col, zeros_pad):
    nchunk = E // GCHUNK
    rows_per_tile = NPAD // 16

    @functools.partial(
        pl.kernel,
        out_type=jax.ShapeDtypeStruct((2 * NPAD, F), jnp.float32),
        mesh=_sc_mesh(),
        scratch_types=[
            pltpu.VMEM((GCHUNK,), jnp.int32),
            pltpu.VMEM((GCHUNK, F), jnp.float32),
            pltpu.VMEM_SHARED((NPAD, F), jnp.float32),
            pltpu.SemaphoreType.DMA,
        ],
    )
    def k(msg_hbm, col_hbm, z_hbm, out_hbm, idx_v, rows_v, acc_sh, sem):
        cidx = lax.axis_index("c")
        sid = lax.axis_index("s")
        wid = sid * 2 + cidx
        rbase = sid * rows_per_tile
        pltpu.sync_copy(z_hbm.at[pl.ds(rbase, rows_per_tile)],
                        acc_sh.at[pl.ds(rbase, rows_per_tile)])
        plsc.subcore_barrier()
        nloop = (nchunk + NTILES - 1) // NTILES

        @pl.loop(0, nloop)
        def _(c):
            cid = wid + NTILES * c

            @pl.when(cid < nchunk)
            def _():
                base = cid * GCHUNK
                pltpu.sync_copy(col_hbm.at[pl.ds(base, GCHUNK)], idx_v)
                pltpu.sync_copy(msg_hbm.at[pl.ds(base, GCHUNK)], rows_v)
                pltpu.sync_copy(rows_v, acc_sh.at[idx_v], add=True)

        plsc.subcore_barrier()
        pltpu.sync_copy(acc_sh.at[pl.ds(rbase, rows_per_tile)],
                        out_hbm.at[pl.ds(cidx * NPAD + rbase, rows_per_tile)])

    return k(msg, col, zeros_pad)


def kernel(x, lin_in_W, lin_in_b, conv0_W1, conv0_b1, conv0_W2, conv0_b2,
           conv0_eps, conv1_W1, conv1_b1, conv1_W2, conv1_b2, conv1_eps,
           lin_out_W, lin_out_b, edge_index):
    T = _tmat()
    xp = jnp.pad(x, ((0, NPAD - N), (0, 0)))
    h = _lin(xp, lin_in_W, lin_in_b.reshape(1, F))
    idx_all = edge_index.reshape(2 * E)
    col = edge_index[1]
    zeros_pad = jnp.zeros((NPAD, F), jnp.float32)
    layers = [
        (conv0_W1, conv0_b1, conv0_W2, conv0_b2, conv0_eps, 6, 6),
        (conv1_W1, conv1_b1, conv1_W2, conv1_b2, conv1_eps, 7, 8),
    ]
    for li, (W1, b1, W2, b2, eps, order, sq) in enumerate(layers):
        w1a = W1[:F]
        w1b = W1[F:]
        eps_arr = jnp.full((8, F), eps, jnp.float32)
        G = _sc_gather(h, idx_all)
        msg = _edge_dense(G, w1a, w1b, b1.reshape(1, 64), W2,
                          b2.reshape(1, F), T, order, sq)
        parts = _sc_scatter(msg, col, zeros_pad)
        if li == 0:
            h = _update(h, parts, eps_arr)
        else:
            h = _update_linout(h, parts, eps_arr, lin_out_W,
                               lin_out_b.reshape(1, F))
    return h[:N]
